# baseline (speedup 1.0000x reference)
import jax
import jax.numpy as jnp
from jax import lax
from jax.experimental import pallas as pl
from jax.experimental.pallas import tpu as pltpu
from jax.experimental.pallas import tpu_sc as plsc

B, C, H, W = 64, 4, 128, 128
P = H * W
NC, NS, L = 2, 16, 16
NW = NC * NS
BPW = B // NW
CHUNK = 4096
NCHUNK = P // CHUNK
ROWS = CHUNK // W
GPR = W // L


def _body(ltrb, cxt, cyt, out, in_v, out_v, cx_v, cy_v, sem):
    wid = lax.axis_index("s") * NC + lax.axis_index("c")
    iota = lax.iota(jnp.int32, L)
    iota4 = iota * 4

    for i in range(BPW):
        b = wid * BPW + i
        pltpu.sync_copy(cxt.at[b], cx_v)
        pltpu.sync_copy(cyt.at[b], cy_v)
        cxs = [cx_v[pl.ds(g * L, L)] for g in range(GPR)]

        for k in range(NCHUNK):
            copies = [
                pltpu.async_copy(
                    ltrb.at[b, c, pl.ds(k * CHUNK, CHUNK)],
                    in_v.at[pl.ds(c * CHUNK, CHUNK)],
                    sem,
                )
                for c in range(C)
            ]
            for cp in copies:
                cp.wait()

            def row_body(r, carry):
                row = k * ROWS + r
                cy = plsc.load_gather(cy_v, [jnp.full((L,), row, jnp.int32)])
                base = r * W
                for g in range(GPR):
                    off = base + g * L
                    lv = in_v[pl.ds(0 * CHUNK + off, L)]
                    tv = in_v[pl.ds(1 * CHUNK + off, L)]
                    rv = in_v[pl.ds(2 * CHUNK + off, L)]
                    bv = in_v[pl.ds(3 * CHUNK + off, L)]
                    plsc.store_scatter(out_v, [iota4 + (off * 4 + 0)], cxs[g] - lv)
                    plsc.store_scatter(out_v, [iota4 + (off * 4 + 1)], cy - tv)
                    plsc.store_scatter(out_v, [iota4 + (off * 4 + 2)], cxs[g] + rv)
                    plsc.store_scatter(out_v, [iota4 + (off * 4 + 3)], cy + bv)
                return carry

            lax.fori_loop(0, ROWS, row_body, 0)
            pltpu.sync_copy(out_v, out.at[b, pl.ds(k * CHUNK * C, CHUNK * C)])


@jax.jit
def _decode(ltrb_flat, cxt, cyt):
    mesh = plsc.VectorSubcoreMesh(
        core_axis_name="c", subcore_axis_name="s", num_cores=NC, num_subcores=NS
    )
    return pl.kernel(
        _body,
        mesh=mesh,
        compiler_params=pltpu.CompilerParams(needs_layout_passes=False),
        out_type=jax.ShapeDtypeStruct((B, P * C), jnp.float32),
        scratch_types=[
            pltpu.VMEM((C * CHUNK,), jnp.float32),
            pltpu.VMEM((C * CHUNK,), jnp.float32),
            pltpu.VMEM((W,), jnp.float32),
            pltpu.VMEM((H,), jnp.float32),
            pltpu.SemaphoreType.DMA,
        ],
    )(ltrb_flat, cxt, cyt)


def kernel(ltrb_map, scales):
    b, c, h, w = ltrb_map.shape
    cyt = (jnp.arange(h, dtype=jnp.float32) + 0.5) * scales[:, 0:1]
    cxt = (jnp.arange(w, dtype=jnp.float32) + 0.5) * scales[:, 1:2]
    out = _decode(ltrb_map.reshape(b, c, h * w), cxt, cyt)
    return out.reshape(b, h * w, c)


# speedup vs baseline: 1.1297x; 1.1297x over previous
import jax
import jax.numpy as jnp
from jax import lax
from jax.experimental import pallas as pl
from jax.experimental.pallas import tpu as pltpu
from jax.experimental.pallas import tpu_sc as plsc

B, C, H, W = 64, 4, 128, 128
P = H * W
NC, NS, L = 2, 16, 16
NW = NC * NS
BPW = B // NW
CHUNK = 4096
NCHUNK = P // CHUNK
NT = BPW * NCHUNK
ROWS = CHUNK // W
GPR = W // L


def _body(ltrb, cxt, cyt, out, in_v, out_v, cx_v, cy_v,
          in_sem0, in_sem1, out_sem0, out_sem1):
    wid = lax.axis_index("s") * NC + lax.axis_index("c")
    b0 = wid * BPW
    iota = lax.iota(jnp.int32, L)
    iota4 = iota * 4
    in_sems = [in_sem0, in_sem1]
    out_sems = [out_sem0, out_sem1]

    for i in range(BPW):
        pltpu.sync_copy(cxt.at[b0 + i], cx_v.at[pl.ds(i * W, W)])
        pltpu.sync_copy(cyt.at[b0 + i], cy_v.at[pl.ds(i * H, H)])
    cxs = [[cx_v[pl.ds(i * W + g * L, L)] for g in range(GPR)]
           for i in range(BPW)]

    def start_in(j):
        i, k = divmod(j, NCHUNK)
        s = j % 2
        return [
            pltpu.async_copy(
                ltrb.at[b0 + i, c, pl.ds(k * CHUNK, CHUNK)],
                in_v.at[pl.ds((s * C + c) * CHUNK, CHUNK)],
                in_sems[s],
            )
            for c in range(C)
        ]

    def start_out(j):
        i, k = divmod(j, NCHUNK)
        s = j % 2
        return pltpu.async_copy(
            out_v.at[pl.ds(s * C * CHUNK, C * CHUNK)],
            out.at[b0 + i, pl.ds(k * CHUNK * C, CHUNK * C)],
            out_sems[s],
        )

    in_flight = {0: start_in(0)}
    out_flight = {}
    for j in range(NT):
        if j + 1 < NT:
            in_flight[j + 1] = start_in(j + 1)
        if j - 2 >= 0:
            out_flight.pop(j - 2).wait()
        for cp in in_flight.pop(j):
            cp.wait()

        i, k = divmod(j, NCHUNK)
        s = j % 2
        ibase = s * C * CHUNK
        obase = s * C * CHUNK
        cxg = cxs[i]

        def row_body(r, carry):
            row = k * ROWS + r
            cy = plsc.load_gather(
                cy_v, [jnp.full((L,), i * H, jnp.int32) + row])
            base = r * W
            for g in range(GPR):
                off = base + g * L
                lv = in_v[pl.ds(ibase + 0 * CHUNK + off, L)]
                tv = in_v[pl.ds(ibase + 1 * CHUNK + off, L)]
                rv = in_v[pl.ds(ibase + 2 * CHUNK + off, L)]
                bv = in_v[pl.ds(ibase + 3 * CHUNK + off, L)]
                o4 = obase + off * 4
                plsc.store_scatter(out_v, [iota4 + (o4 + 0)], cxg[g] - lv)
                plsc.store_scatter(out_v, [iota4 + (o4 + 1)], cy - tv)
                plsc.store_scatter(out_v, [iota4 + (o4 + 2)], cxg[g] + rv)
                plsc.store_scatter(out_v, [iota4 + (o4 + 3)], cy + bv)
            return carry

        lax.fori_loop(0, ROWS, row_body, 0)
        out_flight[j] = start_out(j)

    for j in sorted(out_flight):
        out_flight.pop(j).wait()


@jax.jit
def _decode(ltrb_flat, cxt, cyt):
    mesh = plsc.VectorSubcoreMesh(
        core_axis_name="c", subcore_axis_name="s", num_cores=NC, num_subcores=NS
    )
    return pl.kernel(
        _body,
        mesh=mesh,
        compiler_params=pltpu.CompilerParams(needs_layout_passes=False),
        out_type=jax.ShapeDtypeStruct((B, P * C), jnp.float32),
        scratch_types=[
            pltpu.VMEM((2 * C * CHUNK,), jnp.float32),
            pltpu.VMEM((2 * C * CHUNK,), jnp.float32),
            pltpu.VMEM((BPW * W,), jnp.float32),
            pltpu.VMEM((BPW * H,), jnp.float32),
            pltpu.SemaphoreType.DMA,
            pltpu.SemaphoreType.DMA,
            pltpu.SemaphoreType.DMA,
            pltpu.SemaphoreType.DMA,
        ],
    )(ltrb_flat, cxt, cyt)


def kernel(ltrb_map, scales):
    b, c, h, w = ltrb_map.shape
    cyt = (jnp.arange(h, dtype=jnp.float32) + 0.5) * scales[:, 0:1]
    cxt = (jnp.arange(w, dtype=jnp.float32) + 0.5) * scales[:, 1:2]
    out = _decode(ltrb_map.reshape(b, c, h * w), cxt, cyt)
    return out.reshape(b, h * w, c)


# speedup vs baseline: 2.8103x; 2.4878x over previous
import jax
import jax.numpy as jnp
from jax import lax
from jax.experimental import pallas as pl
from jax.experimental.pallas import tpu as pltpu
from jax.experimental.pallas import tpu_sc as plsc

B, C, H, W = 64, 4, 128, 128
P = H * W
NC, NS, L = 2, 16, 16
NW = NC * NS
BPW = B // NW
ROWS = 32
NCHUNK = H // ROWS
NT = BPW * NCHUNK
GPR = W // L


def _body(ltrb, cxt, cyt, out, in_v, out_v, cx_v, cy_v,
          in_sem0, in_sem1, out_sem0, out_sem1):
    wid = lax.axis_index("s") * NC + lax.axis_index("c")
    b0 = wid * BPW
    in_sems = [in_sem0, in_sem1]
    out_sems = [out_sem0, out_sem1]

    for i in range(BPW):
        pltpu.sync_copy(cxt.at[b0 + i], cx_v.at[pl.ds(i * W, W)])
        pltpu.sync_copy(cyt.at[b0 + i], cy_v.at[pl.ds(i * H, H)])
    cxs = [[cx_v[pl.ds(i * W + g * L, L)] for g in range(GPR)]
           for i in range(BPW)]

    def start_in(j):
        i, k = divmod(j, NCHUNK)
        s = j % 2
        return [
            pltpu.async_copy(
                ltrb.at[b0 + i, c, pl.ds(k * ROWS, ROWS), :],
                in_v.at[s, c],
                in_sems[s],
            )
            for c in range(C)
        ]

    def start_out(j):
        i, k = divmod(j, NCHUNK)
        s = j % 2
        return pltpu.async_copy(
            out_v.at[s],
            out.at[b0 + i, pl.ds(k * ROWS * C, ROWS * C), :],
            out_sems[s],
        )

    in_flight = {0: start_in(0)}
    out_flight = {}
    for j in range(NT):
        if j + 1 < NT:
            in_flight[j + 1] = start_in(j + 1)
        if j - 2 >= 0:
            out_flight.pop(j - 2).wait()
        for cp in in_flight.pop(j):
            cp.wait()

        i, k = divmod(j, NCHUNK)
        s = j % 2
        cxg = cxs[i]

        def row_body(r, carry):
            y = k * ROWS + r
            cy = plsc.load_gather(
                cy_v, [jnp.full((L,), i * H, jnp.int32) + y])
            r4 = r * C
            for g in range(GPR):
                gs = pl.ds(g * L, L)
                lv = in_v[s, 0, r, gs]
                tv = in_v[s, 1, r, gs]
                rv = in_v[s, 2, r, gs]
                bv = in_v[s, 3, r, gs]
                out_v[s, r4 + 0, gs] = cxg[g] - lv
                out_v[s, r4 + 1, gs] = cy - tv
                out_v[s, r4 + 2, gs] = cxg[g] + rv
                out_v[s, r4 + 3, gs] = cy + bv
            return carry

        lax.fori_loop(0, ROWS, row_body, 0)
        out_flight[j] = start_out(j)

    for j in sorted(out_flight):
        out_flight.pop(j).wait()


@jax.jit
def _decode(ltrb_map, cxt, cyt):
    mesh = plsc.VectorSubcoreMesh(
        core_axis_name="c", subcore_axis_name="s", num_cores=NC, num_subcores=NS
    )
    return pl.kernel(
        _body,
        mesh=mesh,
        compiler_params=pltpu.CompilerParams(needs_layout_passes=False),
        out_type=jax.ShapeDtypeStruct((B, H * C, W), jnp.float32),
        scratch_types=[
            pltpu.VMEM((2, C, ROWS, W), jnp.float32),
            pltpu.VMEM((2, ROWS * C, W), jnp.float32),
            pltpu.VMEM((BPW * W,), jnp.float32),
            pltpu.VMEM((BPW * H,), jnp.float32),
            pltpu.SemaphoreType.DMA,
            pltpu.SemaphoreType.DMA,
            pltpu.SemaphoreType.DMA,
            pltpu.SemaphoreType.DMA,
        ],
    )(ltrb_map, cxt, cyt)


def kernel(ltrb_map, scales):
    b, c, h, w = ltrb_map.shape
    cyt = (jnp.arange(h, dtype=jnp.float32) + 0.5) * scales[:, 0:1]
    cxt = (jnp.arange(w, dtype=jnp.float32) + 0.5) * scales[:, 1:2]
    out = _decode(ltrb_map, cxt, cyt)
    return out.reshape(b, h, c, w).transpose(0, 1, 3, 2).reshape(b, h * w, c)


# speedup vs baseline: 2.9821x; 1.0611x over previous
import jax
import jax.numpy as jnp
from jax import lax
from jax.experimental import pallas as pl
from jax.experimental.pallas import tpu as pltpu
from jax.experimental.pallas import tpu_sc as plsc

B, C, H, W = 64, 4, 128, 128
P = H * W
NC, NS, L = 2, 16, 16
NW = NC * NS
BPW = B // NW
ROWS = 32
NCHUNK = H // ROWS
NT = BPW * NCHUNK
GPR = W // L


def _body(ltrb, cxt, cyt, out, in_v, out_v, cx_v, cy_v,
          in_sem0, in_sem1, out_sem0, out_sem1):
    wid = lax.axis_index("s") * NC + lax.axis_index("c")
    b0 = wid * BPW
    in_sems = [in_sem0, in_sem1]
    out_sems = [out_sem0, out_sem1]

    for i in range(BPW):
        pltpu.sync_copy(cxt.at[b0 + i], cx_v.at[pl.ds(i * W, W)])
        pltpu.sync_copy(cyt.at[b0 + i], cy_v.at[pl.ds(i * H, H)])
    def start_in_dyn(j, s):
        i = j // NCHUNK
        k = j % NCHUNK
        for c in range(C):
            pltpu.async_copy(
                ltrb.at[b0 + i, c, pl.ds(k * ROWS, ROWS), :],
                in_v.at[s, c],
                in_sems[s],
            )

    def wait_in(s):
        for c in range(C):
            pltpu.make_async_copy(
                ltrb.at[b0, c, pl.ds(0, ROWS), :], in_v.at[s, c], in_sems[s]
            ).wait()

    def start_out_dyn(j, s):
        i = j // NCHUNK
        k = j % NCHUNK
        pltpu.async_copy(
            out_v.at[s],
            out.at[b0 + i, pl.ds(k * ROWS * C, ROWS * C), :],
            out_sems[s],
        )

    def wait_out(s):
        pltpu.make_async_copy(
            out_v.at[s], out.at[b0, pl.ds(0, ROWS * C), :], out_sems[s]
        ).wait()

    start_in_dyn(0, 0)
    start_in_dyn(1, 1)

    def outer_body(t, carry):
        for s in range(2):
            j = 2 * t + s

            @pl.when(t > 0)
            def _drain_out():
                wait_out(s)

            wait_in(s)

            i = j // NCHUNK
            k = j % NCHUNK
            cxg = [cx_v[pl.ds(i * W + g * L, L)] for g in range(GPR)]

            def row_body(r, carry2):
                cy = plsc.load_gather(
                    cy_v, [jnp.full((L,), i * H + k * ROWS + r, jnp.int32)])
                r4 = r * C
                for g in range(GPR):
                    gs = pl.ds(g * L, L)
                    lv = in_v[s, 0, r, gs]
                    tv = in_v[s, 1, r, gs]
                    rv = in_v[s, 2, r, gs]
                    bv = in_v[s, 3, r, gs]
                    out_v[s, r4 + 0, gs] = cxg[g] - lv
                    out_v[s, r4 + 1, gs] = cy - tv
                    out_v[s, r4 + 2, gs] = cxg[g] + rv
                    out_v[s, r4 + 3, gs] = cy + bv
                return carry2

            lax.fori_loop(0, ROWS, row_body, 0)

            @pl.when(t < NT // 2 - 1)
            def _prefetch():
                start_in_dyn(j + 2, s)

            start_out_dyn(j, s)
        return carry

    lax.fori_loop(0, NT // 2, outer_body, 0)
    for s in range(2):
        wait_out(s)


@jax.jit
def _decode(ltrb_map, cxt, cyt):
    mesh = plsc.VectorSubcoreMesh(
        core_axis_name="c", subcore_axis_name="s", num_cores=NC, num_subcores=NS
    )
    return pl.kernel(
        _body,
        mesh=mesh,
        compiler_params=pltpu.CompilerParams(needs_layout_passes=False),
        out_type=jax.ShapeDtypeStruct((B, H * C, W), jnp.float32),
        scratch_types=[
            pltpu.VMEM((2, C, ROWS, W), jnp.float32),
            pltpu.VMEM((2, ROWS * C, W), jnp.float32),
            pltpu.VMEM((BPW * W,), jnp.float32),
            pltpu.VMEM((BPW * H,), jnp.float32),
            pltpu.SemaphoreType.DMA,
            pltpu.SemaphoreType.DMA,
            pltpu.SemaphoreType.DMA,
            pltpu.SemaphoreType.DMA,
        ],
    )(ltrb_map, cxt, cyt)


def kernel(ltrb_map, scales):
    b, c, h, w = ltrb_map.shape
    cyt = (jnp.arange(h, dtype=jnp.float32) + 0.5) * scales[:, 0:1]
    cxt = (jnp.arange(w, dtype=jnp.float32) + 0.5) * scales[:, 1:2]
    out = _decode(ltrb_map, cxt, cyt)
    return out.reshape(b, h, c, w).transpose(0, 1, 3, 2).reshape(b, h * w, c)


# speedup vs baseline: 3.2084x; 1.0759x over previous
import jax
import jax.numpy as jnp
from jax import lax
from jax.experimental import pallas as pl
from jax.experimental.pallas import tpu as pltpu
from jax.experimental.pallas import tpu_sc as plsc

B, C, H, W = 64, 4, 128, 128
P = H * W
NC, NS, L = 2, 16, 16
NW = NC * NS
BPW = B // NW
ROWS = 32
NCHUNK = H // ROWS
NT = BPW * NCHUNK
GPR = W // L


def _body(ltrb, cxt, cyt, out, in_v, out_v, cx_v, cy_v,
          in_sem0, in_sem1, out_sem0, out_sem1):
    wid = lax.axis_index("s") * NC + lax.axis_index("c")
    b0 = wid * BPW
    in_sems = [in_sem0, in_sem1]
    out_sems = [out_sem0, out_sem1]

    for i in range(BPW):
        pltpu.sync_copy(cxt.at[b0 + i], cx_v.at[pl.ds(i * W, W)])
        pltpu.sync_copy(cyt.at[b0 + i], cy_v.at[pl.ds(i * H, H)])
    def start_in_dyn(j, s):
        i = j // NCHUNK
        k = j % NCHUNK
        for c in range(C):
            pltpu.async_copy(
                ltrb.at[b0 + i, c, pl.ds(k * ROWS, ROWS), :],
                in_v.at[s, c],
                in_sems[s],
            )

    def wait_in(s):
        for c in range(C):
            pltpu.make_async_copy(
                ltrb.at[b0, c, pl.ds(0, ROWS), :], in_v.at[s, c], in_sems[s]
            ).wait()

    def start_out_dyn(j, s):
        i = j // NCHUNK
        k = j % NCHUNK
        pltpu.async_copy(
            out_v.at[s],
            out.at[b0 + i, pl.ds(k * ROWS * C, ROWS * C), :],
            out_sems[s],
        )

    def wait_out(s):
        pltpu.make_async_copy(
            out_v.at[s], out.at[b0, pl.ds(0, ROWS * C), :], out_sems[s]
        ).wait()

    start_in_dyn(0, 0)
    start_in_dyn(1, 1)

    def outer_body(t, carry):
        for s in range(2):
            j = 2 * t + s

            @pl.when(t > 0)
            def _drain_out():
                wait_out(s)

            wait_in(s)

            i = j // NCHUNK
            k = j % NCHUNK
            cxg = [cx_v[pl.ds(i * W + g * L, L)] for g in range(GPR)]

            @plsc.parallel_loop(0, ROWS, unroll=2)
            def row_body(r):
                cy = plsc.load_gather(
                    cy_v, [jnp.full((L,), i * H + k * ROWS + r, jnp.int32)])
                r4 = r * C
                for g in range(GPR):
                    gs = pl.ds(g * L, L)
                    lv = in_v[s, 0, r, gs]
                    tv = in_v[s, 1, r, gs]
                    rv = in_v[s, 2, r, gs]
                    bv = in_v[s, 3, r, gs]
                    out_v[s, r4 + 0, gs] = cxg[g] - lv
                    out_v[s, r4 + 1, gs] = cy - tv
                    out_v[s, r4 + 2, gs] = cxg[g] + rv
                    out_v[s, r4 + 3, gs] = cy + bv

            @pl.when(t < NT // 2 - 1)
            def _prefetch():
                start_in_dyn(j + 2, s)

            start_out_dyn(j, s)
        return carry

    lax.fori_loop(0, NT // 2, outer_body, 0)
    for s in range(2):
        wait_out(s)


@jax.jit
def _decode(ltrb_map, cxt, cyt):
    mesh = plsc.VectorSubcoreMesh(
        core_axis_name="c", subcore_axis_name="s", num_cores=NC, num_subcores=NS
    )
    return pl.kernel(
        _body,
        mesh=mesh,
        compiler_params=pltpu.CompilerParams(needs_layout_passes=False),
        out_type=jax.ShapeDtypeStruct((B, H * C, W), jnp.float32),
        scratch_types=[
            pltpu.VMEM((2, C, ROWS, W), jnp.float32),
            pltpu.VMEM((2, ROWS * C, W), jnp.float32),
            pltpu.VMEM((BPW * W,), jnp.float32),
            pltpu.VMEM((BPW * H,), jnp.float32),
            pltpu.SemaphoreType.DMA,
            pltpu.SemaphoreType.DMA,
            pltpu.SemaphoreType.DMA,
            pltpu.SemaphoreType.DMA,
        ],
    )(ltrb_map, cxt, cyt)


def kernel(ltrb_map, scales):
    b, c, h, w = ltrb_map.shape
    cyt = (jnp.arange(h, dtype=jnp.float32) + 0.5) * scales[:, 0:1]
    cxt = (jnp.arange(w, dtype=jnp.float32) + 0.5) * scales[:, 1:2]
    out = _decode(ltrb_map, cxt, cyt)
    return out.reshape(b, h, c, w).transpose(0, 1, 3, 2).reshape(b, h * w, c)
